# baseline (device time: 23419 ns/iter reference)
import jax
import jax.numpy as jnp
from jax import lax
from jax.experimental import pallas as pl
from jax.experimental.pallas import tpu as pltpu

N_DEV = 4
N_TOK = 512
D_IN = 256
H_OUT = 512
N_EXP = 16
E_LOCAL = 4
M_BLK = N_TOK // N_DEV


def kernel(x, router_W, route_idx, expert_W):
    def body(x_ref, rw_ref, idx_ref, ew_ref, out_ref,
             partial_ref, send_buf, recv_buf, send_sems, recv_sems):
        my = lax.axis_index("i")
        left = lax.rem(my + N_DEV - 1, N_DEV)
        right = lax.rem(my + 1, N_DEV)

        barrier_sem = pltpu.get_barrier_semaphore()
        for nbr in (left, right):
            pl.semaphore_signal(
                barrier_sem, inc=1,
                device_id=(nbr,), device_id_type=pl.DeviceIdType.MESH,
            )
        pl.semaphore_wait(barrier_sem, 2)

        xv = x_ref[:, :]
        scores = jnp.dot(xv, rw_ref[:, :], preferred_element_type=jnp.float32)
        s_max = jnp.max(scores, axis=-1, keepdims=True)
        probs = jnp.exp(scores - s_max)
        probs = probs / jnp.sum(probs, axis=-1, keepdims=True)

        e0 = idx_ref[:, 0:1]
        e1 = idx_ref[:, 1:2]
        ids = lax.broadcasted_iota(jnp.int32, (N_TOK, N_EXP), 1)
        top2 = (ids == e0) | (ids == e1)
        masked = jnp.where(top2, probs, 0.0)
        coeff = masked / jnp.sum(masked, axis=-1, keepdims=True)

        rows = lax.broadcasted_iota(jnp.int32, (N_EXP, E_LOCAL), 0)
        cols = lax.broadcasted_iota(jnp.int32, (N_EXP, E_LOCAL), 1)
        onehot = (rows == E_LOCAL * my + cols).astype(jnp.float32)
        coeff_local = jnp.dot(coeff, onehot,
                              preferred_element_type=jnp.float32)

        acc = jnp.zeros((N_TOK, H_OUT), dtype=jnp.float32)
        for le in range(E_LOCAL):
            y = jnp.dot(xv, ew_ref[le], preferred_element_type=jnp.float32)
            acc = acc + coeff_local[:, le:le + 1] * y
        partial_ref[:, :] = acc

        for t in range(N_DEV - 1):
            s_idx = lax.rem(my + 2 * N_DEV - 1 - t, N_DEV)
            if t == 0:
                send_buf[:, :] = partial_ref[pl.ds(s_idx * M_BLK, M_BLK), :]
            else:
                send_buf[:, :] = (recv_buf[t - 1]
                                  + partial_ref[pl.ds(s_idx * M_BLK, M_BLK), :])
            rdma = pltpu.make_async_remote_copy(
                src_ref=send_buf,
                dst_ref=recv_buf.at[t],
                send_sem=send_sems.at[t],
                recv_sem=recv_sems.at[t],
                device_id=(right,),
                device_id_type=pl.DeviceIdType.MESH,
            )
            rdma.start()
            rdma.wait()

        out_ref[:, :] = (recv_buf[N_DEV - 2]
                         + partial_ref[pl.ds(my * M_BLK, M_BLK), :])

    return pl.pallas_call(
        body,
        out_shape=jax.ShapeDtypeStruct((M_BLK, H_OUT), jnp.float32),
        in_specs=[
            pl.BlockSpec(memory_space=pltpu.VMEM),
            pl.BlockSpec(memory_space=pltpu.VMEM),
            pl.BlockSpec(memory_space=pltpu.VMEM),
            pl.BlockSpec(memory_space=pltpu.VMEM),
        ],
        out_specs=pl.BlockSpec(memory_space=pltpu.VMEM),
        scratch_shapes=[
            pltpu.VMEM((N_TOK, H_OUT), jnp.float32),
            pltpu.VMEM((M_BLK, H_OUT), jnp.float32),
            pltpu.VMEM((N_DEV - 1, M_BLK, H_OUT), jnp.float32),
            pltpu.SemaphoreType.DMA((N_DEV - 1,)),
            pltpu.SemaphoreType.DMA((N_DEV - 1,)),
        ],
        compiler_params=pltpu.CompilerParams(collective_id=0),
    )(x, router_W, route_idx, expert_W)


# device time: 17329 ns/iter; 1.3514x vs baseline; 1.3514x over previous
import jax
import jax.numpy as jnp
from jax import lax
from jax.experimental import pallas as pl
from jax.experimental.pallas import tpu as pltpu

N_DEV = 4
N_TOK = 512
D_IN = 256
H_OUT = 512
N_EXP = 16
E_LOCAL = 4
M_BLK = N_TOK // N_DEV


def kernel(x, router_W, route_idx, expert_W):
    def body(x_ref, rw_ref, idx_ref, ew_ref, out_ref,
             coeff_ref, send_bufs, recv_buf, send_sems, recv_sems):
        my = lax.axis_index("i")
        peers = [lax.rem(my + 1 + t, N_DEV) for t in range(N_DEV - 1)]

        barrier_sem = pltpu.get_barrier_semaphore()
        for p in peers:
            pl.semaphore_signal(
                barrier_sem, inc=1,
                device_id=(p,), device_id_type=pl.DeviceIdType.MESH,
            )
        pl.semaphore_wait(barrier_sem, N_DEV - 1)

        scores = jnp.dot(x_ref[:, :], rw_ref[:, :],
                         preferred_element_type=jnp.float32)
        s_max = jnp.max(scores, axis=-1, keepdims=True)
        probs = jnp.exp(scores - s_max)
        probs = probs / jnp.sum(probs, axis=-1, keepdims=True)

        e0 = idx_ref[:, 0:1]
        e1 = idx_ref[:, 1:2]
        ids = lax.broadcasted_iota(jnp.int32, (N_TOK, N_EXP), 1)
        top2 = (ids == e0) | (ids == e1)
        masked = jnp.where(top2, probs, 0.0)
        coeff = masked / jnp.sum(masked, axis=-1, keepdims=True)

        rows = lax.broadcasted_iota(jnp.int32, (N_EXP, E_LOCAL), 0)
        cols = lax.broadcasted_iota(jnp.int32, (N_EXP, E_LOCAL), 1)
        onehot = (rows == E_LOCAL * my + cols).astype(jnp.float32)
        coeff_ref[:, :] = jnp.dot(coeff, onehot,
                                  preferred_element_type=jnp.float32)

        def block_partial(k):
            xb = x_ref[pl.ds(k * M_BLK, M_BLK), :]
            cb = coeff_ref[pl.ds(k * M_BLK, M_BLK), :]
            acc = jnp.zeros((M_BLK, H_OUT), dtype=jnp.float32)
            for le in range(E_LOCAL):
                y = jnp.dot(xb, ew_ref[le], preferred_element_type=jnp.float32)
                acc = acc + cb[:, le:le + 1] * y
            return acc

        send_rdmas = []
        for t, k in enumerate(peers):
            send_bufs[t] = block_partial(k)
            rdma = pltpu.make_async_remote_copy(
                src_ref=send_bufs.at[t],
                dst_ref=recv_buf.at[my],
                send_sem=send_sems.at[t],
                recv_sem=recv_sems.at[my],
                device_id=(k,),
                device_id_type=pl.DeviceIdType.MESH,
            )
            rdma.start()
            send_rdmas.append(rdma)

        recv_buf[my] = block_partial(my)

        for p in peers:
            pltpu.make_async_remote_copy(
                src_ref=recv_buf.at[p],
                dst_ref=recv_buf.at[p],
                send_sem=send_sems.at[0],
                recv_sem=recv_sems.at[p],
                device_id=(p,),
                device_id_type=pl.DeviceIdType.MESH,
            ).wait_recv()

        out_ref[:, :] = (recv_buf[0] + recv_buf[1]
                         + recv_buf[2] + recv_buf[3])

        for rdma in send_rdmas:
            rdma.wait_send()

    return pl.pallas_call(
        body,
        out_shape=jax.ShapeDtypeStruct((M_BLK, H_OUT), jnp.float32),
        in_specs=[
            pl.BlockSpec(memory_space=pltpu.VMEM),
            pl.BlockSpec(memory_space=pltpu.VMEM),
            pl.BlockSpec(memory_space=pltpu.VMEM),
            pl.BlockSpec(memory_space=pltpu.VMEM),
        ],
        out_specs=pl.BlockSpec(memory_space=pltpu.VMEM),
        scratch_shapes=[
            pltpu.VMEM((N_TOK, E_LOCAL), jnp.float32),
            pltpu.VMEM((N_DEV - 1, M_BLK, H_OUT), jnp.float32),
            pltpu.VMEM((N_DEV, M_BLK, H_OUT), jnp.float32),
            pltpu.SemaphoreType.DMA((N_DEV - 1,)),
            pltpu.SemaphoreType.DMA((N_DEV,)),
        ],
        compiler_params=pltpu.CompilerParams(collective_id=0),
    )(x, router_W, route_idx, expert_W)


# device time: 12871 ns/iter; 1.8195x vs baseline; 1.3464x over previous
import jax
import jax.numpy as jnp
from jax import lax
from jax.experimental import pallas as pl
from jax.experimental.pallas import tpu as pltpu

N_DEV = 4
N_TOK = 512
D_IN = 256
H_OUT = 512
N_EXP = 16
E_LOCAL = 4
M_BLK = N_TOK // N_DEV


def kernel(x, router_W, route_idx, expert_W):
    ids16 = jnp.arange(N_EXP, dtype=route_idx.dtype)
    top2f = jnp.any(
        route_idx[:, :, None] == ids16[None, None, :], axis=1
    ).astype(x.dtype)
    packed = (
        jnp.pad(x, ((0, N_EXP), (0, N_EXP)))
        + jnp.pad(top2f, ((0, N_EXP), (D_IN, 0)))
        + jnp.pad(router_W.T, ((N_TOK, 0), (0, N_EXP)))
    )

    def body(packed_ref, ew_ref, out_hbm,
             out_stage, cl_ref, send_bufs, recv_buf,
             out_sem, send_sems, recv_sems):
        my = lax.axis_index("i")
        peers = [lax.rem(my + 1 + t, N_DEV) for t in range(N_DEV - 1)]

        barrier_sem = pltpu.get_barrier_semaphore()
        for p in peers:
            pl.semaphore_signal(
                barrier_sem, inc=1,
                device_id=(p,), device_id_type=pl.DeviceIdType.MESH,
            )

        xv = packed_ref[0:N_TOK, 0:D_IN]
        rwt = packed_ref[N_TOK:N_TOK + N_EXP, 0:D_IN]
        scores = lax.dot_general(
            xv, rwt,
            dimension_numbers=(((1,), (1,)), ((), ())),
            preferred_element_type=jnp.float32,
        )
        s_max = jnp.max(scores, axis=-1, keepdims=True)
        probs = jnp.exp(scores - s_max)
        probs = probs / jnp.sum(probs, axis=-1, keepdims=True)

        mask = packed_ref[0:N_TOK, D_IN:D_IN + N_EXP]
        masked = probs * mask
        coeff = masked / jnp.sum(masked, axis=-1, keepdims=True)

        rows = lax.broadcasted_iota(jnp.int32, (N_EXP, E_LOCAL), 0)
        cols = lax.broadcasted_iota(jnp.int32, (N_EXP, E_LOCAL), 1)
        onehot = (rows == E_LOCAL * my + cols).astype(jnp.float32)
        cl_ref[:, :] = jnp.dot(coeff, onehot,
                               preferred_element_type=jnp.float32)

        def block_partial(k):
            xb = packed_ref[pl.ds(k * M_BLK, M_BLK), 0:D_IN]
            cb = cl_ref[pl.ds(k * M_BLK, M_BLK), :]
            acc = jnp.zeros((M_BLK, H_OUT), dtype=jnp.float32)
            for le in range(E_LOCAL):
                y = jnp.dot(xb, ew_ref[le],
                            preferred_element_type=jnp.float32)
                acc = acc + cb[:, le:le + 1] * y
            return acc

        send_rdmas = []
        for t, k in enumerate(peers):
            send_bufs[t] = block_partial(k).astype(jnp.bfloat16)
            if t == 0:
                pl.semaphore_wait(barrier_sem, N_DEV - 1)
            rdma = pltpu.make_async_remote_copy(
                src_ref=send_bufs.at[t],
                dst_ref=recv_buf.at[my],
                send_sem=send_sems.at[t],
                recv_sem=recv_sems.at[my],
                device_id=(k,),
                device_id_type=pl.DeviceIdType.MESH,
            )
            rdma.start()
            send_rdmas.append(rdma)

        acc = block_partial(my)

        for t in range(N_DEV - 1):
            p = lax.rem(my + N_DEV - 1 - t, N_DEV)
            pltpu.make_async_remote_copy(
                src_ref=recv_buf.at[p],
                dst_ref=recv_buf.at[p],
                send_sem=send_sems.at[0],
                recv_sem=recv_sems.at[p],
                device_id=(p,),
                device_id_type=pl.DeviceIdType.MESH,
            ).wait_recv()
            acc = acc + recv_buf[p].astype(jnp.float32)

        out_stage[:, :] = acc
        cp_out = pltpu.make_async_copy(out_stage, out_hbm, out_sem)
        cp_out.start()
        cp_out.wait()

        for rdma in send_rdmas:
            rdma.wait_send()

    out = pl.pallas_call(
        body,
        out_shape=jax.ShapeDtypeStruct((M_BLK, H_OUT), jnp.float32),
        in_specs=[
            pl.BlockSpec(memory_space=pltpu.VMEM),
            pl.BlockSpec(memory_space=pltpu.VMEM),
        ],
        out_specs=pl.BlockSpec(memory_space=pl.ANY),
        scratch_shapes=[
            pltpu.VMEM((M_BLK, H_OUT), jnp.float32),
            pltpu.VMEM((N_TOK, E_LOCAL), jnp.float32),
            pltpu.VMEM((N_DEV - 1, M_BLK, H_OUT), jnp.bfloat16),
            pltpu.VMEM((N_DEV, M_BLK, H_OUT), jnp.bfloat16),
            pltpu.SemaphoreType.DMA,
            pltpu.SemaphoreType.DMA((N_DEV - 1,)),
            pltpu.SemaphoreType.DMA((N_DEV,)),
        ],
        compiler_params=pltpu.CompilerParams(collective_id=0),
    )(packed, expert_W)
    return out
